# baseline (speedup 1.0000x reference)
import functools

import jax
import jax.numpy as jnp
from jax import lax
from jax.experimental import pallas as pl
from jax.experimental.pallas import tpu as pltpu
from jax.experimental.pallas import tpu_sc as plsc

N_NODES = 10000
N_EDGES = 320000
D_IN = 128
H1 = 16
H2 = 10

NUM_CORES = 2
NUM_SUBCORES = 16
NUM_WORKERS = NUM_CORES * NUM_SUBCORES
EDGES_PER_WORKER = N_EDGES // NUM_WORKERS
CHUNK = 128
SUB = 1
SUPER = SUB * CHUNK
NUM_CHUNKS = (EDGES_PER_WORKER + SUPER - 1) // SUPER
EDGES_PAD = NUM_CHUNKS * SUPER
REAL_VECS = EDGES_PER_WORKER // 16
N_PAD = 10240
ROWS_PER_TILE = N_PAD // NUM_SUBCORES
ZCHUNK = 128

_f32 = jnp.float32


def _splat(v, lane):
  idx = jnp.full((16, 1), lane, dtype=jnp.int32)
  dnums = lax.GatherDimensionNumbers(
      offset_dims=(), collapsed_slice_dims=(0,), start_index_map=(0,))
  return lax.gather(v, idx, dnums, (1,),
                    mode=lax.GatherScatterMode.PROMISE_IN_BOUNDS)


def _make_edge_kernel(with_denom):
  p_width = 16
  DEPTH = 4
  mesh = plsc.VectorSubcoreMesh(
      core_axis_name="c", subcore_axis_name="s",
      num_cores=NUM_CORES, num_subcores=NUM_SUBCORES)

  out_type = [jax.ShapeDtypeStruct((NUM_CORES, N_PAD, p_width), _f32)]
  scratch = [
      pltpu.VMEM((N_NODES,), _f32),
      pltpu.VMEM((N_NODES,), _f32),
      pltpu.VMEM((NUM_CHUNKS, SUPER), jnp.int32),
      pltpu.VMEM((NUM_CHUNKS, SUPER), jnp.int32),
      pltpu.VMEM((SUPER, p_width), _f32),
      pltpu.VMEM((SUPER, p_width), _f32),
      pltpu.VMEM((SUPER, p_width), _f32),
      pltpu.VMEM((SUPER, p_width), _f32),
      pltpu.VMEM((SUPER, p_width), _f32),
      pltpu.VMEM((SUPER, p_width), _f32),
      pltpu.VMEM((ZCHUNK, p_width), _f32),
      pltpu.VMEM_SHARED((N_PAD, p_width), _f32),
      pltpu.SemaphoreType.DMA,
      pltpu.SemaphoreType.DMA,
      pltpu.SemaphoreType.DMA,
      pltpu.SemaphoreType.DMA,
      pltpu.SemaphoreType.DMA,
      pltpu.SemaphoreType.DMA,
  ]
  if with_denom:
    out_type.append(jax.ShapeDtypeStruct((NUM_CORES, N_PAD), _f32))
    scratch += [
        pltpu.VMEM((SUPER,), _f32),
        pltpu.VMEM((SUPER,), _f32),
        pltpu.VMEM((ROWS_PER_TILE,), _f32),
        pltpu.VMEM_SHARED((N_PAD,), _f32),
        pltpu.SemaphoreType.DMA,
        pltpu.SemaphoreType.DMA,
    ]

  @functools.partial(
      pl.kernel,
      mesh=mesh,
      compiler_params=pltpu.CompilerParams(
          needs_layout_passes=False, use_tc_tiling_on_sc=False),
      out_type=tuple(out_type) if with_denom else out_type[0],
      scratch_types=scratch,
  )
  def edge_kernel(h_hbm, as_hbm, ad_hbm, src_hbm, dst_hbm, *out_and_scratch):
    if with_denom:
      (out_hbm, dout_hbm,
       as_v, ad_v, src_v, dst_v, g0_v, g1_v, g2_v, g3_v, s0_v, s1_v,
       zero_v, agg_sp, sem_g0, sem_g1, sem_g2, sem_g3, sem_s0, sem_s1,
       w0_v, w1_v, zd_v, den_sp, sem_d0, sem_d1) = out_and_scratch
    else:
      (out_hbm,
       as_v, ad_v, src_v, dst_v, g0_v, g1_v, g2_v, g3_v, s0_v, s1_v,
       zero_v, agg_sp, sem_g0, sem_g1, sem_g2, sem_g3, sem_s0,
       sem_s1) = out_and_scratch
    gbufs = ((g0_v, sem_g0), (g1_v, sem_g1), (g2_v, sem_g2), (g3_v, sem_g3))
    c = lax.axis_index("c")
    s = lax.axis_index("s")
    wid = c * NUM_SUBCORES + s

    cp_as = pltpu.async_copy(as_hbm, as_v, sem_g0)
    cp_ad = pltpu.async_copy(ad_hbm, ad_v, sem_g1)
    cp_src = pltpu.async_copy(src_hbm.at[wid], src_v, sem_s0)
    cp_dst = pltpu.async_copy(dst_hbm.at[wid], dst_v, sem_s1)

    zvec = jnp.zeros((16,), _f32)

    def zero_body(r, carry):
      zero_v[r, pl.ds(0, 16)] = zvec
      return carry

    lax.fori_loop(0, ZCHUNK, zero_body, 0)
    if with_denom:
      def zd_body(i, carry):
        zd_v[pl.ds(i * 16, 16)] = zvec
        return carry

      lax.fori_loop(0, ROWS_PER_TILE // 16, zd_body, 0)
    cp_src.wait()
    cp_dst.wait()
    cp_as.wait()
    cp_ad.wait()
    for k in range(ROWS_PER_TILE // ZCHUNK):
      pltpu.sync_copy(
          zero_v, agg_sp.at[pl.ds(s * ROWS_PER_TILE + k * ZCHUNK, ZCHUNK)])
    if with_denom:
      pltpu.sync_copy(
          zd_v, den_sp.at[pl.ds(s * ROWS_PER_TILE, ROWS_PER_TILE)])
    plsc.subcore_barrier()

    def start_gather(j, buf, sem):
      pltpu.async_copy(h_hbm.at[src_v.at[j]], buf, sem)

    def wait_gather(buf, sem):
      pltpu.make_async_copy(h_hbm.at[src_v.at[0]], buf, sem).wait()

    def start_scatter(j, buf, sem):
      pltpu.async_copy(buf, agg_sp.at[dst_v.at[j]], sem, add=True)

    def wait_scatter(buf, sem):
      pltpu.make_async_copy(buf, agg_sp.at[dst_v.at[0]], sem).wait()

    def start_dscatter(j, wbuf, sem):
      pltpu.async_copy(wbuf, den_sp.at[dst_v.at[j]], sem, add=True)

    def wait_dscatter(wbuf, sem):
      pltpu.make_async_copy(wbuf, den_sp.at[dst_v.at[0]], sem).wait()

    def process(j, gbuf, sbuf, wbuf):
      for g in range(SUPER // 16):
        si = src_v[j, pl.ds(g * 16, 16)]
        di = dst_v[j, pl.ds(g * 16, 16)]
        e = plsc.load_gather(as_v, [si]) + plsc.load_gather(ad_v, [di])
        e = jnp.where(e >= 0.0, e, 0.2 * e)
        wv = jnp.exp(e)
        wv = jnp.where(
            jnp.broadcast_to(j * (SUPER // 16) + g < REAL_VECS, (16,)),
            wv, 0.0)
        if with_denom:
          wbuf[pl.ds(g * 16, 16)] = wv
        for l in range(16):
          sc = _splat(wv, l)
          row = g * 16 + l
          sbuf[row, pl.ds(0, 16)] = gbuf[row, pl.ds(0, 16)] * sc

    sbufs = ((s0_v, sem_s0), (s1_v, sem_s1))
    if with_denom:
      wbufs = ((w0_v, sem_d0), (w1_v, sem_d1))
    else:
      wbufs = ((None, None), (None, None))
    for k in range(DEPTH):
      start_gather(k, *gbufs[k])

    n_quads = (NUM_CHUNKS - 3) // DEPTH

    def quad_body(jj, carry):
      for k in range(DEPTH):
        c = DEPTH * jj + k
        gbuf, gsem = gbufs[k]
        sbuf, ssem = sbufs[k % 2]
        wbuf, wsem = wbufs[k % 2]
        wait_gather(gbuf, gsem)

        if k < 2:
          @pl.when(jj > 0)
          def _():
            wait_scatter(sbuf, ssem)
            if with_denom:
              wait_dscatter(wbuf, wsem)
        else:
          wait_scatter(sbuf, ssem)
          if with_denom:
            wait_dscatter(wbuf, wsem)

        process(c, gbuf, sbuf, wbuf)
        start_scatter(c, sbuf, ssem)
        if with_denom:
          start_dscatter(c, wbuf, wsem)

        @pl.when(c + DEPTH < NUM_CHUNKS)
        def _():
          start_gather(c + DEPTH, gbuf, gsem)
      return carry

    lax.fori_loop(0, n_quads, quad_body, 0)

    for tail in range(DEPTH * n_quads, NUM_CHUNKS):
      gbuf, gsem = gbufs[tail % DEPTH]
      sbuf, ssem = sbufs[tail % 2]
      wbuf, wsem = wbufs[tail % 2]
      wait_gather(gbuf, gsem)
      wait_scatter(sbuf, ssem)
      if with_denom:
        wait_dscatter(wbuf, wsem)
      process(tail, gbuf, sbuf, wbuf)
      start_scatter(tail, sbuf, ssem)
      if with_denom:
        start_dscatter(tail, wbuf, wsem)

    for drain_buf, drain_sem in sbufs:
      wait_scatter(drain_buf, drain_sem)
    if with_denom:
      for drain_buf, drain_sem in wbufs:
        wait_dscatter(drain_buf, drain_sem)
    plsc.subcore_barrier()

    pltpu.sync_copy(
        agg_sp.at[pl.ds(s * ROWS_PER_TILE, ROWS_PER_TILE)],
        out_hbm.at[c].at[pl.ds(s * ROWS_PER_TILE, ROWS_PER_TILE)])
    if with_denom:
      pltpu.sync_copy(
          den_sp.at[pl.ds(s * ROWS_PER_TILE, ROWS_PER_TILE)],
          dout_hbm.at[c].at[pl.ds(s * ROWS_PER_TILE, ROWS_PER_TILE)])

  return edge_kernel


_edge_kernel_l1 = _make_edge_kernel(True)
_edge_kernel_l2 = _make_edge_kernel(False)


def _tc_layer1(x, w1, a1s, a1d):
  def body(x_ref, w_ref, s_ref, d_ref, h_ref, as_ref, ad_ref):
    h = jnp.dot(x_ref[...], w_ref[...], preferred_element_type=_f32)
    as_ref[...] = jnp.sum(h * s_ref[...], axis=1, keepdims=True)
    ad_ref[...] = jnp.sum(h * d_ref[...], axis=1, keepdims=True)
    h_ref[...] = h

  return pl.pallas_call(
      body,
      out_shape=(
          jax.ShapeDtypeStruct((N_NODES, 16), _f32),
          jax.ShapeDtypeStruct((N_NODES, 1), _f32),
          jax.ShapeDtypeStruct((N_NODES, 1), _f32),
      ),
  )(x, w1, a1s, a1d)


def _tc_layer2(parts, dparts, b1, w2, a2s, a2d):
  def body(p_ref, dp_ref, b_ref, w_ref, s_ref, d_ref, h_ref, as_ref, ad_ref):
    raw = p_ref[0, :N_NODES] + p_ref[1, :N_NODES]
    den = dp_ref[0, :N_NODES] + dp_ref[1, :N_NODES]
    agg = raw / (den + 1e-16)
    z = jnp.maximum(agg + b_ref[...], 0.0)
    h2 = jnp.dot(z, w_ref[...], preferred_element_type=_f32)
    as_ref[...] = jnp.sum(h2 * s_ref[...], axis=1, keepdims=True)
    ad_ref[...] = jnp.sum(h2 * d_ref[...], axis=1, keepdims=True)
    h_ref[...] = jnp.concatenate(
        [h2, jnp.ones((N_NODES, 1), _f32), jnp.zeros((N_NODES, 5), _f32)],
        axis=1)

  return pl.pallas_call(
      body,
      out_shape=(
          jax.ShapeDtypeStruct((N_NODES, 16), _f32),
          jax.ShapeDtypeStruct((N_NODES, 1), _f32),
          jax.ShapeDtypeStruct((N_NODES, 1), _f32),
      ),
  )(parts, dparts, b1, w2, a2s, a2d)


def _tc_head(parts, b2, l1w, l1b, l2w, l2b):
  def body(p_ref, b_ref, w1_ref, b1_ref, w2_ref, b2_ref, o_ref):
    raw = p_ref[0, :N_NODES] + p_ref[1, :N_NODES]
    agg = raw[:, :H2] / (raw[:, H2:H2 + 1] + 1e-16)
    y = agg + b_ref[...]
    t = jnp.maximum(
        jnp.dot(y, w1_ref[...], preferred_element_type=_f32) + b1_ref[...],
        0.0)
    o_ref[...] = (
        jnp.dot(t, w2_ref[...], preferred_element_type=_f32) + b2_ref[...])

  return pl.pallas_call(
      body,
      out_shape=jax.ShapeDtypeStruct((N_NODES, 1), _f32),
  )(parts, b2, l1w, l1b, l2w, l2b)


def kernel(x, edge_index, W1, a1_src, a1_dst, b1, W2, a2_src, a2_dst, b2,
           lin1_W, lin1_b, lin2_W, lin2_b):
  src = edge_index[0].reshape(NUM_WORKERS, EDGES_PER_WORKER)
  dst = edge_index[1].reshape(NUM_WORKERS, EDGES_PER_WORKER)
  pad = EDGES_PAD - EDGES_PER_WORKER
  src = jnp.pad(src, ((0, 0), (0, pad))).reshape(
      NUM_WORKERS, NUM_CHUNKS, SUPER)
  dst = jnp.pad(dst, ((0, 0), (0, pad))).reshape(
      NUM_WORKERS, NUM_CHUNKS, SUPER)

  h1, as1, ad1 = _tc_layer1(x, W1, a1_src.reshape(1, H1),
                            a1_dst.reshape(1, H1))
  parts1, dparts1 = _edge_kernel_l1(h1, as1.reshape(N_NODES),
                                    ad1.reshape(N_NODES), src, dst)
  h2, as2, ad2 = _tc_layer2(parts1, dparts1.reshape(NUM_CORES, N_PAD, 1),
                            b1.reshape(1, H1),
                            W2, a2_src.reshape(1, H2), a2_dst.reshape(1, H2))
  parts2 = _edge_kernel_l2(h2, as2.reshape(N_NODES), ad2.reshape(N_NODES),
                           src, dst)
  return _tc_head(parts2, b2.reshape(1, H2), lin1_W, lin1_b.reshape(1, H2),
                  lin2_W, lin2_b.reshape(1, 1))


# speedup vs baseline: 1.0030x; 1.0030x over previous
import functools

import jax
import jax.numpy as jnp
from jax import lax
from jax.experimental import pallas as pl
from jax.experimental.pallas import tpu as pltpu
from jax.experimental.pallas import tpu_sc as plsc

N_NODES = 10000
N_EDGES = 320000
D_IN = 128
H1 = 16
H2 = 10

NUM_CORES = 2
NUM_SUBCORES = 16
NUM_WORKERS = NUM_CORES * NUM_SUBCORES
EDGES_PER_WORKER = N_EDGES // NUM_WORKERS
CHUNK = 128
SUB = 1
SUPER = SUB * CHUNK
NUM_CHUNKS = (EDGES_PER_WORKER + SUPER - 1) // SUPER
EDGES_PAD = NUM_CHUNKS * SUPER
REAL_VECS = EDGES_PER_WORKER // 16
N_PAD = 10240
ROWS_PER_TILE = N_PAD // NUM_SUBCORES
ZCHUNK = 128

_f32 = jnp.float32


def _splat(v, lane):
  idx = jnp.full((16, 1), lane, dtype=jnp.int32)
  dnums = lax.GatherDimensionNumbers(
      offset_dims=(), collapsed_slice_dims=(0,), start_index_map=(0,))
  return lax.gather(v, idx, dnums, (1,),
                    mode=lax.GatherScatterMode.PROMISE_IN_BOUNDS)


def _make_edge_kernel(with_denom):
  p_width = 16
  DEPTH = 4
  mesh = plsc.VectorSubcoreMesh(
      core_axis_name="c", subcore_axis_name="s",
      num_cores=NUM_CORES, num_subcores=NUM_SUBCORES)

  out_type = [jax.ShapeDtypeStruct((NUM_CORES, N_PAD, p_width), _f32)]
  scratch = [
      pltpu.VMEM((N_NODES,), _f32),
      pltpu.VMEM((N_NODES,), _f32),
      pltpu.VMEM((NUM_CHUNKS, SUPER), jnp.int32),
      pltpu.VMEM((NUM_CHUNKS, SUPER), jnp.int32),
      pltpu.VMEM((SUPER, p_width), _f32),
      pltpu.VMEM((SUPER, p_width), _f32),
      pltpu.VMEM((SUPER, p_width), _f32),
      pltpu.VMEM((SUPER, p_width), _f32),
      pltpu.VMEM((SUPER, p_width), _f32),
      pltpu.VMEM((SUPER, p_width), _f32),
      pltpu.VMEM((ZCHUNK, p_width), _f32),
      pltpu.VMEM_SHARED((N_PAD, p_width), _f32),
      pltpu.SemaphoreType.DMA,
      pltpu.SemaphoreType.DMA,
      pltpu.SemaphoreType.DMA,
      pltpu.SemaphoreType.DMA,
      pltpu.SemaphoreType.DMA,
      pltpu.SemaphoreType.DMA,
  ]
  if with_denom:
    out_type.append(jax.ShapeDtypeStruct((NUM_CORES, N_PAD), _f32))
    scratch += [
        pltpu.VMEM((SUPER,), _f32),
        pltpu.VMEM((SUPER,), _f32),
        pltpu.VMEM((ROWS_PER_TILE,), _f32),
        pltpu.VMEM_SHARED((N_PAD,), _f32),
        pltpu.SemaphoreType.DMA,
        pltpu.SemaphoreType.DMA,
    ]

  @functools.partial(
      pl.kernel,
      mesh=mesh,
      compiler_params=pltpu.CompilerParams(
          needs_layout_passes=False, use_tc_tiling_on_sc=False),
      out_type=tuple(out_type) if with_denom else out_type[0],
      scratch_types=scratch,
  )
  def edge_kernel(h_hbm, as_hbm, ad_hbm, src_hbm, dst_hbm, *out_and_scratch):
    if with_denom:
      (out_hbm, dout_hbm,
       as_v, ad_v, src_v, dst_v, g0_v, g1_v, g2_v, g3_v, s0_v, s1_v,
       zero_v, agg_sp, sem_g0, sem_g1, sem_g2, sem_g3, sem_s0, sem_s1,
       w0_v, w1_v, zd_v, den_sp, sem_d0, sem_d1) = out_and_scratch
    else:
      (out_hbm,
       as_v, ad_v, src_v, dst_v, g0_v, g1_v, g2_v, g3_v, s0_v, s1_v,
       zero_v, agg_sp, sem_g0, sem_g1, sem_g2, sem_g3, sem_s0,
       sem_s1) = out_and_scratch
    gbufs = ((g0_v, sem_g0), (g1_v, sem_g1), (g2_v, sem_g2), (g3_v, sem_g3))
    c = lax.axis_index("c")
    s = lax.axis_index("s")
    wid = c * NUM_SUBCORES + s

    cp_as = pltpu.async_copy(as_hbm, as_v, sem_g0)
    cp_ad = pltpu.async_copy(ad_hbm, ad_v, sem_g1)
    cp_src = pltpu.async_copy(src_hbm.at[wid], src_v, sem_s0)
    cp_dst = pltpu.async_copy(dst_hbm.at[wid], dst_v, sem_s1)

    zvec = jnp.zeros((16,), _f32)

    def zero_body(r, carry):
      zero_v[r, pl.ds(0, 16)] = zvec
      return carry

    lax.fori_loop(0, ZCHUNK, zero_body, 0)
    if with_denom:
      def zd_body(i, carry):
        zd_v[pl.ds(i * 16, 16)] = zvec
        return carry

      lax.fori_loop(0, ROWS_PER_TILE // 16, zd_body, 0)
    cp_src.wait()
    cp_dst.wait()
    cp_as.wait()
    cp_ad.wait()
    for k in range(ROWS_PER_TILE // ZCHUNK):
      pltpu.sync_copy(
          zero_v, agg_sp.at[pl.ds(s * ROWS_PER_TILE + k * ZCHUNK, ZCHUNK)])
    if with_denom:
      pltpu.sync_copy(
          zd_v, den_sp.at[pl.ds(s * ROWS_PER_TILE, ROWS_PER_TILE)])
    plsc.subcore_barrier()

    def start_gather(j, buf, sem):
      pltpu.async_copy(h_hbm.at[src_v.at[j]], buf, sem)

    def wait_gather(buf, sem):
      pltpu.make_async_copy(h_hbm.at[src_v.at[0]], buf, sem).wait()

    def start_scatter(j, buf, sem):
      pltpu.async_copy(buf, agg_sp.at[dst_v.at[j]], sem, add=True)

    def wait_scatter(buf, sem):
      pltpu.make_async_copy(buf, agg_sp.at[dst_v.at[0]], sem).wait()

    def start_dscatter(j, wbuf, sem):
      pltpu.async_copy(wbuf, den_sp.at[dst_v.at[j]], sem, add=True)

    def wait_dscatter(wbuf, sem):
      pltpu.make_async_copy(wbuf, den_sp.at[dst_v.at[0]], sem).wait()

    def process(j, gbuf, sbuf, wbuf, masked=False):
      for g in range(SUPER // 16):
        si = src_v[j, pl.ds(g * 16, 16)]
        di = dst_v[j, pl.ds(g * 16, 16)]
        e = plsc.load_gather(as_v, [si]) + plsc.load_gather(ad_v, [di])
        wv = jnp.exp(jnp.maximum(e, 0.2 * e))
        if masked and (NUM_CHUNKS - 1) * (SUPER // 16) + g >= REAL_VECS:
          wv = jnp.zeros((16,), _f32)
        if with_denom:
          wbuf[pl.ds(g * 16, 16)] = wv
        for l in range(16):
          sc = _splat(wv, l)
          row = g * 16 + l
          sbuf[row, pl.ds(0, 16)] = gbuf[row, pl.ds(0, 16)] * sc

    sbufs = ((s0_v, sem_s0), (s1_v, sem_s1))
    if with_denom:
      wbufs = ((w0_v, sem_d0), (w1_v, sem_d1))
    else:
      wbufs = ((None, None), (None, None))
    for k in range(DEPTH):
      start_gather(k, *gbufs[k])

    n_quads = (NUM_CHUNKS - 3) // DEPTH

    def quad_body(jj, carry):
      for k in range(DEPTH):
        c = DEPTH * jj + k
        gbuf, gsem = gbufs[k]
        sbuf, ssem = sbufs[k % 2]
        wbuf, wsem = wbufs[k % 2]
        wait_gather(gbuf, gsem)

        if k < 2:
          @pl.when(jj > 0)
          def _():
            wait_scatter(sbuf, ssem)
            if with_denom:
              wait_dscatter(wbuf, wsem)
        else:
          wait_scatter(sbuf, ssem)
          if with_denom:
            wait_dscatter(wbuf, wsem)

        process(c, gbuf, sbuf, wbuf)
        start_scatter(c, sbuf, ssem)
        if with_denom:
          start_dscatter(c, wbuf, wsem)

        @pl.when(c + DEPTH < NUM_CHUNKS)
        def _():
          start_gather(c + DEPTH, gbuf, gsem)
      return carry

    lax.fori_loop(0, n_quads, quad_body, 0)

    for tail in range(DEPTH * n_quads, NUM_CHUNKS):
      gbuf, gsem = gbufs[tail % DEPTH]
      sbuf, ssem = sbufs[tail % 2]
      wbuf, wsem = wbufs[tail % 2]
      wait_gather(gbuf, gsem)
      wait_scatter(sbuf, ssem)
      if with_denom:
        wait_dscatter(wbuf, wsem)
      process(tail, gbuf, sbuf, wbuf, masked=(tail == NUM_CHUNKS - 1))
      start_scatter(tail, sbuf, ssem)
      if with_denom:
        start_dscatter(tail, wbuf, wsem)

    for drain_buf, drain_sem in sbufs:
      wait_scatter(drain_buf, drain_sem)
    if with_denom:
      for drain_buf, drain_sem in wbufs:
        wait_dscatter(drain_buf, drain_sem)
    plsc.subcore_barrier()

    pltpu.sync_copy(
        agg_sp.at[pl.ds(s * ROWS_PER_TILE, ROWS_PER_TILE)],
        out_hbm.at[c].at[pl.ds(s * ROWS_PER_TILE, ROWS_PER_TILE)])
    if with_denom:
      pltpu.sync_copy(
          den_sp.at[pl.ds(s * ROWS_PER_TILE, ROWS_PER_TILE)],
          dout_hbm.at[c].at[pl.ds(s * ROWS_PER_TILE, ROWS_PER_TILE)])

  return edge_kernel


_edge_kernel_l1 = _make_edge_kernel(True)
_edge_kernel_l2 = _make_edge_kernel(False)


def _tc_layer1(x, w1, a1s, a1d):
  def body(x_ref, w_ref, s_ref, d_ref, h_ref, as_ref, ad_ref):
    h = jnp.dot(x_ref[...], w_ref[...], preferred_element_type=_f32)
    as_ref[...] = jnp.sum(h * s_ref[...], axis=1, keepdims=True)
    ad_ref[...] = jnp.sum(h * d_ref[...], axis=1, keepdims=True)
    h_ref[...] = h

  return pl.pallas_call(
      body,
      out_shape=(
          jax.ShapeDtypeStruct((N_NODES, 16), _f32),
          jax.ShapeDtypeStruct((N_NODES, 1), _f32),
          jax.ShapeDtypeStruct((N_NODES, 1), _f32),
      ),
  )(x, w1, a1s, a1d)


def _tc_layer2(parts, dparts, b1, w2, a2s, a2d):
  def body(p_ref, dp_ref, b_ref, w_ref, s_ref, d_ref, h_ref, as_ref, ad_ref):
    raw = p_ref[0, :N_NODES] + p_ref[1, :N_NODES]
    den = dp_ref[0, :N_NODES] + dp_ref[1, :N_NODES]
    agg = raw / (den + 1e-16)
    z = jnp.maximum(agg + b_ref[...], 0.0)
    h2 = jnp.dot(z, w_ref[...], preferred_element_type=_f32)
    as_ref[...] = jnp.sum(h2 * s_ref[...], axis=1, keepdims=True)
    ad_ref[...] = jnp.sum(h2 * d_ref[...], axis=1, keepdims=True)
    h_ref[...] = jnp.concatenate(
        [h2, jnp.ones((N_NODES, 1), _f32), jnp.zeros((N_NODES, 5), _f32)],
        axis=1)

  return pl.pallas_call(
      body,
      out_shape=(
          jax.ShapeDtypeStruct((N_NODES, 16), _f32),
          jax.ShapeDtypeStruct((N_NODES, 1), _f32),
          jax.ShapeDtypeStruct((N_NODES, 1), _f32),
      ),
  )(parts, dparts, b1, w2, a2s, a2d)


def _tc_head(parts, b2, l1w, l1b, l2w, l2b):
  def body(p_ref, b_ref, w1_ref, b1_ref, w2_ref, b2_ref, o_ref):
    raw = p_ref[0, :N_NODES] + p_ref[1, :N_NODES]
    agg = raw[:, :H2] / (raw[:, H2:H2 + 1] + 1e-16)
    y = agg + b_ref[...]
    t = jnp.maximum(
        jnp.dot(y, w1_ref[...], preferred_element_type=_f32) + b1_ref[...],
        0.0)
    o_ref[...] = (
        jnp.dot(t, w2_ref[...], preferred_element_type=_f32) + b2_ref[...])

  return pl.pallas_call(
      body,
      out_shape=jax.ShapeDtypeStruct((N_NODES, 1), _f32),
  )(parts, b2, l1w, l1b, l2w, l2b)


def kernel(x, edge_index, W1, a1_src, a1_dst, b1, W2, a2_src, a2_dst, b2,
           lin1_W, lin1_b, lin2_W, lin2_b):
  src = edge_index[0].reshape(NUM_WORKERS, EDGES_PER_WORKER)
  dst = edge_index[1].reshape(NUM_WORKERS, EDGES_PER_WORKER)
  pad = EDGES_PAD - EDGES_PER_WORKER
  src = jnp.pad(src, ((0, 0), (0, pad))).reshape(
      NUM_WORKERS, NUM_CHUNKS, SUPER)
  dst = jnp.pad(dst, ((0, 0), (0, pad))).reshape(
      NUM_WORKERS, NUM_CHUNKS, SUPER)

  h1, as1, ad1 = _tc_layer1(x, W1, a1_src.reshape(1, H1),
                            a1_dst.reshape(1, H1))
  parts1, dparts1 = _edge_kernel_l1(h1, as1.reshape(N_NODES),
                                    ad1.reshape(N_NODES), src, dst)
  h2, as2, ad2 = _tc_layer2(parts1, dparts1.reshape(NUM_CORES, N_PAD, 1),
                            b1.reshape(1, H1),
                            W2, a2_src.reshape(1, H2), a2_dst.reshape(1, H2))
  parts2 = _edge_kernel_l2(h2, as2.reshape(N_NODES), ad2.reshape(N_NODES),
                           src, dst)
  return _tc_head(parts2, b2.reshape(1, H2), lin1_W, lin1_b.reshape(1, H2),
                  lin2_W, lin2_b.reshape(1, 1))
